# baseline (device time: 46361 ns/iter reference)
import jax
import jax.numpy as jnp
from jax import lax
from jax.experimental import pallas as pl
from jax.experimental.pallas import tpu as pltpu

N_DEV = 4
N_HALF = 2


def kernel(x):
    m, n = x.shape[1], x.shape[2]
    x2 = x.reshape(m, n)
    half = m // N_HALF
    qr = half // N_DEV

    def body(x_ref, out_ref, rbuf, send_sems, recv_sems):
        i = lax.axis_index("i")
        px = i // 2
        py = (i % 2 + i // 2) % 2
        pa = i + 1 - 2 * (i % 2)
        pb = 3 - i

        plan = (
            ((pa, py), (pb, px)),
            ((pb, px), (pa, py)),
        )

        barrier = pltpu.get_barrier_semaphore()
        for b in (pa, pb):
            pl.semaphore_signal(
                barrier, inc=1,
                device_id=(b,), device_id_type=pl.DeviceIdType.MESH,
            )
        pl.semaphore_wait(barrier, 2)

        def qrows(h, half_bit, quarter_bit):
            return pl.ds(h * half + half_bit * qr * 2 + quarter_bit * qr, qr)

        def cast(rows):
            out_ref[rows, :] = x_ref[rows, :].astype(out_ref.dtype)

        descs = {}

        def start(h, s, src_rows, target, to_rbuf):
            src = out_ref.at[src_rows, :]
            dst = rbuf.at[h, s] if to_rbuf else src
            rdma = pltpu.make_async_remote_copy(
                src_ref=src,
                dst_ref=dst,
                send_sem=send_sems.at[h, s],
                recv_sem=recv_sems.at[h, s],
                device_id=(target,),
                device_id_type=pl.DeviceIdType.MESH,
            )
            rdma.start()
            descs[(h, s)] = rdma

        for sub in range(2):
            for h in range(N_HALF):
                (p1, b1), (p2, b2) = plan[h]
                q = (1 - b2) if sub == 0 else b2
                rows = qrows(h, 1 - b1, q)
                cast(rows)
                start(h, sub, rows, p1, to_rbuf=True)

        for sub in range(2):
            for h in range(N_HALF):
                (p1, b1), (p2, b2) = plan[h]
                cast(qrows(h, b1, (1 - b2) if sub == 0 else b2))

        for h in range(N_HALF):
            (p1, b1), (p2, b2) = plan[h]
            descs[(h, 0)].wait_recv()
            acc = out_ref.at[qrows(h, b1, 1 - b2), :]
            acc[...] = acc[...] + rbuf[h, 0]
            start(h, 2, qrows(h, b1, 1 - b2), p2, to_rbuf=True)

        for h in range(N_HALF):
            (p1, b1), (p2, b2) = plan[h]
            descs[(h, 1)].wait_recv()
            acc = out_ref.at[qrows(h, b1, b2), :]
            acc[...] = acc[...] + rbuf[h, 1]

        for h in range(N_HALF):
            (p1, b1), (p2, b2) = plan[h]
            descs[(h, 2)].wait_recv()
            acc = out_ref.at[qrows(h, b1, b2), :]
            acc[...] = acc[...] + rbuf[h, 2]
            start(h, 3, qrows(h, b1, b2), p2, to_rbuf=False)
            start(h, 4, qrows(h, b1, b2), p1, to_rbuf=False)

        for h in range(N_HALF):
            (p1, b1), (p2, b2) = plan[h]
            descs[(h, 3)].wait_recv()
            start(h, 5, qrows(h, b1, 1 - b2), p1, to_rbuf=False)

        for h in range(N_HALF):
            descs[(h, 4)].wait_recv()
            descs[(h, 5)].wait_recv()

        for r in descs.values():
            r.wait_send()

    return pl.pallas_call(
        body,
        out_shape=jax.ShapeDtypeStruct((m, n), jnp.bfloat16),
        in_specs=[pl.BlockSpec(memory_space=pltpu.VMEM)],
        out_specs=pl.BlockSpec(memory_space=pltpu.VMEM),
        scratch_shapes=[
            pltpu.VMEM((N_HALF, 3, qr, n), jnp.bfloat16),
            pltpu.SemaphoreType.DMA((N_HALF, 6)),
            pltpu.SemaphoreType.DMA((N_HALF, 6)),
        ],
        compiler_params=pltpu.CompilerParams(collective_id=0),
    )(x2)


# device time: 44405 ns/iter; 1.0440x vs baseline; 1.0440x over previous
import jax
import jax.numpy as jnp
from jax import lax
from jax.experimental import pallas as pl
from jax.experimental.pallas import tpu as pltpu

N_DEV = 4
N_DIR = 2
K_SUB = 2


def kernel(x):
    m, n = x.shape[1], x.shape[2]
    x2 = x.reshape(m, n)
    half = m // N_DIR
    ch = half // N_DEV
    sub = ch // K_SUB

    def body(x_ref, out_ref, rbuf, send_sems, recv_sems):
        i = lax.axis_index("i")
        right = (i + 1) % N_DEV
        left = (i - 1) % N_DEV
        nbr = (right, left)

        barrier = pltpu.get_barrier_semaphore()
        for b in (left, right):
            pl.semaphore_signal(
                barrier, inc=1,
                device_id=(b,), device_id_type=pl.DeviceIdType.MESH,
            )
        pl.semaphore_wait(barrier, 2)

        descs = {}

        def sub_slice(d, c, j):
            return pl.ds(d * half + c * ch + j * sub, sub)

        def start(d, s, j, c, to_rbuf):
            src = out_ref.at[sub_slice(d, c, j), :]
            dst = rbuf.at[d, s, j] if to_rbuf else src
            rdma = pltpu.make_async_remote_copy(
                src_ref=src,
                dst_ref=dst,
                send_sem=send_sems.at[d, s, j],
                recv_sem=recv_sems.at[d, s, j],
                device_id=(nbr[d],),
                device_id_type=pl.DeviceIdType.MESH,
            )
            rdma.start()
            descs[(d, s, j)] = rdma

        for c_off in range(N_DEV):
            c = (i + c_off) % N_DEV
            for d in range(N_DIR):
                rows = pl.ds(d * half + c * ch, ch)
                out_ref[rows, :] = x_ref[rows, :].astype(out_ref.dtype)
            if c_off == 0:
                for j in range(K_SUB):
                    for d in range(N_DIR):
                        start(d, 0, j, i, to_rbuf=True)

        def dir_order(s, j):
            return (0, 1) if (s + j) % 2 == 0 else (1, 0)

        for s in range(1, N_DEV - 1):
            for j in range(K_SUB):
                for d in dir_order(s, j):
                    descs[(d, s - 1, j)].wait_recv()
                    c = (i - s) % N_DEV if d == 0 else (i + s) % N_DEV
                    acc = out_ref.at[sub_slice(d, c, j), :]
                    acc[...] = acc[...] + rbuf[d, s - 1, j]
                    start(d, s, j, c, to_rbuf=True)

        for j in range(K_SUB):
            for d in dir_order(3, j):
                descs[(d, N_DEV - 2, j)].wait_recv()
                c = (i + 1) % N_DEV if d == 0 else (i + 3) % N_DEV
                acc = out_ref.at[sub_slice(d, c, j), :]
                acc[...] = acc[...] + rbuf[d, N_DEV - 2, j]
                start(d, 3, j, c, to_rbuf=False)

        for s in range(1, N_DEV - 1):
            for j in range(K_SUB):
                for d in dir_order(s, j):
                    descs[(d, 3 + s - 1, j)].wait_recv()
                    c = (i + 1 - s) % N_DEV if d == 0 else (i + 3 + s) % N_DEV
                    start(d, 3 + s, j, c, to_rbuf=False)

        for j in range(K_SUB):
            for d in range(N_DIR):
                descs[(d, 5, j)].wait_recv()
        for r in descs.values():
            r.wait_send()

    return pl.pallas_call(
        body,
        out_shape=jax.ShapeDtypeStruct((m, n), jnp.bfloat16),
        in_specs=[pl.BlockSpec(memory_space=pltpu.VMEM)],
        out_specs=pl.BlockSpec(memory_space=pltpu.VMEM),
        scratch_shapes=[
            pltpu.VMEM((N_DIR, N_DEV - 1, K_SUB, sub, n), jnp.bfloat16),
            pltpu.SemaphoreType.DMA((N_DIR, 2 * (N_DEV - 1), K_SUB)),
            pltpu.SemaphoreType.DMA((N_DIR, 2 * (N_DEV - 1), K_SUB)),
        ],
        compiler_params=pltpu.CompilerParams(collective_id=0),
    )(x2)
